# baseline (speedup 1.0000x reference)
import jax
import jax.numpy as jnp
from jax import lax
from jax.experimental import pallas as pl
from jax.experimental.pallas import tpu as pltpu

_XS = 128
_K = 8192
_B = 4096
_DIM = 256
_BK = 1024
_NKB = _K // _BK
_NJB = 2
_BJ = _B // _NJB


def _som_body(x_ref, s_ref, bmu_ref, dists_ref,
              xt_ref, b2_ref, rmin_ref, ridx_ref):
    i = pl.program_id(0)
    j = pl.program_id(1)
    first = (i == 0) & (j == 0)

    @pl.when(first)
    def _():
        xtv = jnp.swapaxes(x_ref[...], 0, 1)
        xt_ref[...] = xtv
        t = xtv * xtv
        size = _DIM
        while size > 8:
            h = size // 2
            t = t[:h, :] + t[h:size, :]
            size = h
        b2_ref[...] = jnp.sum(t, axis=0, keepdims=True)

    cols = pl.ds(j * _BJ, _BJ)
    s = s_ref[...]
    a2 = jnp.sum(s * s, axis=1, keepdims=True)
    prod = lax.dot_general(s * jnp.float32(-2.0), xt_ref[:, cols],
                           (((1,), (0,)), ((), ())),
                           preferred_element_type=jnp.float32)
    d2 = jnp.maximum(a2 + b2_ref[:, cols] + prod, 1e-12)
    dists_ref[...] = d2 * lax.rsqrt(d2)

    half = _BK // 2
    rows = lax.broadcasted_iota(jnp.int32, (half, _BJ), 0)
    va, vb = d2[:half, :], d2[half:, :]
    upd = vb < va
    val = jnp.minimum(va, vb)
    idx = jnp.where(upd, rows + half, rows)
    size = half
    while size > 8:
        h = size // 2
        upd = val[h:size, :] < val[:h, :]
        idx = jnp.where(upd, idx[h:size, :], idx[:h, :])
        val = jnp.minimum(val[:h, :], val[h:size, :])
        size = h
    lmin = jnp.min(val, axis=0, keepdims=True)
    lidx = jnp.min(jnp.where(val == lmin, idx, jnp.int32(2**30)),
                   axis=0, keepdims=True)

    @pl.when(i == 0)
    def _():
        rmin_ref[:, cols] = lmin
        ridx_ref[:, cols] = lidx

    @pl.when(i > 0)
    def _():
        better = lmin < rmin_ref[:, cols]
        ridx_ref[:, cols] = jnp.where(better, lidx + i * _BK,
                                      ridx_ref[:, cols])
        rmin_ref[:, cols] = jnp.where(better, lmin, rmin_ref[:, cols])

    @pl.when(i == _NKB - 1)
    def _():
        idx = ridx_ref[:, cols]
        bmu_ref[0:1, cols] = idx // _XS
        bmu_ref[1:2, cols] = idx % _XS


def kernel(x, somap):
    bmu2, dists = pl.pallas_call(
        _som_body,
        grid=(_NKB, _NJB),
        in_specs=[
            pl.BlockSpec((_B, _DIM), lambda i, j: (0, 0)),
            pl.BlockSpec((_BK, _DIM), lambda i, j: (i, 0)),
        ],
        out_specs=[
            pl.BlockSpec((2, _B), lambda i, j: (0, 0)),
            pl.BlockSpec((_BK, _BJ), lambda i, j: (i, j)),
        ],
        out_shape=[
            jax.ShapeDtypeStruct((2, _B), jnp.int32),
            jax.ShapeDtypeStruct((_K, _B), jnp.float32),
        ],
        scratch_shapes=[
            pltpu.VMEM((_DIM, _B), jnp.float32),
            pltpu.VMEM((1, _B), jnp.float32),
            pltpu.VMEM((1, _B), jnp.float32),
            pltpu.VMEM((1, _B), jnp.int32),
        ],
    )(x, somap)
    return bmu2.T, dists


# speedup vs baseline: 1.0170x; 1.0170x over previous
import jax
import jax.numpy as jnp
from jax import lax
from jax.experimental import pallas as pl
from jax.experimental.pallas import tpu as pltpu

_XS = 128
_K = 8192
_B = 4096
_DIM = 256
_BK = 1024
_NKB = _K // _BK


def _som_body(x_ref, s_ref, bmu_ref, dists_ref,
              xt_ref, b2_ref, rmin_ref, ridx_ref):
    i = pl.program_id(0)

    @pl.when(i == 0)
    def _():
        xtv = jnp.swapaxes(x_ref[...], 0, 1)
        xt_ref[...] = xtv
        t = xtv * xtv
        size = _DIM
        while size > 8:
            h = size // 2
            t = t[:h, :] + t[h:size, :]
            size = h
        b2_ref[...] = jnp.sum(t, axis=0, keepdims=True)

    s = s_ref[...]
    a2 = jnp.sum(s * s, axis=1, keepdims=True)
    prod = lax.dot_general(s * jnp.float32(-2.0), xt_ref[...],
                           (((1,), (0,)), ((), ())),
                           preferred_element_type=jnp.float32)
    d2 = jnp.maximum(a2 + b2_ref[...] + prod, 1e-12)
    dists_ref[...] = d2 * lax.rsqrt(d2)

    half = _BK // 2
    rows = lax.broadcasted_iota(jnp.int32, (half, _B), 0)
    va, vb = d2[:half, :], d2[half:, :]
    upd = vb < va
    val = jnp.minimum(va, vb)
    idx = jnp.where(upd, rows + half, rows)
    size = half
    while size > 8:
        h = size // 2
        upd = val[h:size, :] < val[:h, :]
        idx = jnp.where(upd, idx[h:size, :], idx[:h, :])
        val = jnp.minimum(val[:h, :], val[h:size, :])
        size = h
    lmin = jnp.min(val, axis=0, keepdims=True)
    lidx = jnp.min(jnp.where(val == lmin, idx, jnp.int32(2**30)),
                   axis=0, keepdims=True)

    @pl.when(i == 0)
    def _():
        rmin_ref[...] = lmin
        ridx_ref[...] = lidx

    @pl.when(i > 0)
    def _():
        better = lmin < rmin_ref[...]
        ridx_ref[...] = jnp.where(better, lidx + i * _BK, ridx_ref[...])
        rmin_ref[...] = jnp.where(better, lmin, rmin_ref[...])

    @pl.when(i == _NKB - 1)
    def _():
        idx = ridx_ref[...]
        bmu_ref[0:1, :] = idx // _XS
        bmu_ref[1:2, :] = idx % _XS


def kernel(x, somap):
    bmu2, dists = pl.pallas_call(
        _som_body,
        grid=(_NKB,),
        in_specs=[
            pl.BlockSpec((_B, _DIM), lambda i: (0, 0)),
            pl.BlockSpec((_BK, _DIM), lambda i: (i, 0)),
        ],
        out_specs=[
            pl.BlockSpec((2, _B), lambda i: (0, 0)),
            pl.BlockSpec((_BK, _B), lambda i: (i, 0)),
        ],
        out_shape=[
            jax.ShapeDtypeStruct((2, _B), jnp.int32),
            jax.ShapeDtypeStruct((_K, _B), jnp.float32),
        ],
        scratch_shapes=[
            pltpu.VMEM((_DIM, _B), jnp.float32),
            pltpu.VMEM((1, _B), jnp.float32),
            pltpu.VMEM((1, _B), jnp.float32),
            pltpu.VMEM((1, _B), jnp.int32),
        ],
    )(x, somap)
    return bmu2.T, dists
